# baseline (speedup 1.0000x reference)
import functools

import jax
import jax.numpy as jnp
from jax import lax
from jax.experimental import pallas as pl
from jax.experimental.pallas import tpu as pltpu
from jax.experimental.pallas import tpu_sc as plsc

N = 50000
E = 800000
IN_DIM = 771
HID = 64
H = 2
DH = 32
G = 64

NC = 2
NS = 16
LANES = 16

EPT = E // NS
CHUNK = 80
NCHUNK = EPT // CHUNK
NPT = 3128
N_PAD = NPT * NS

BN = 2000
GRID = N // BN


def _proj_stores(h1, kqvW_ref, kqvb_ref, arelB_ref, mrelB_ref, qsc_ref,
                 qT_ref, kvT_ref):
    kqv = jnp.dot(h1, kqvW_ref[...], preferred_element_type=jnp.float32) + kqvb_ref[...]
    k = jnp.dot(kqv[:, 0:HID], arelB_ref[...], preferred_element_type=jnp.float32)
    q = kqv[:, HID:2 * HID] * qsc_ref[...]
    v = jnp.dot(kqv[:, 2 * HID:3 * HID], mrelB_ref[...], preferred_element_type=jnp.float32)
    qT_ref[0] = q[:, 0:DH]
    qT_ref[1] = q[:, DH:2 * DH]
    kvT_ref[0, :, 0:DH] = k[:, 0:DH]
    kvT_ref[0, :, DH:2 * DH] = v[:, 0:DH]
    kvT_ref[1, :, 0:DH] = k[:, DH:2 * DH]
    kvT_ref[1, :, DH:2 * DH] = v[:, DH:2 * DH]


def _k1_body(x_ref, w_ref, b_ref, kqvW_ref, kqvb_ref, arelB_ref, mrelB_ref,
             qsc_ref, h_ref, qT_ref, kvT_ref):
    h = jnp.dot(x_ref[...], w_ref[...], preferred_element_type=jnp.float32)
    h = jnp.maximum(h + b_ref[...], 0.0)
    h_ref[...] = h
    _proj_stores(h, kqvW_ref, kqvb_ref, arelB_ref, mrelB_ref, qsc_ref,
                 qT_ref, kvT_ref)


def _post_layer(h_ref, U_ref, den_ref, aW_ref, ab_ref, beta_ref):
    n0 = U_ref[0] / (den_ref[0] + 1e-16)
    n1 = U_ref[1] / (den_ref[1] + 1e-16)
    agg = jnp.concatenate([n0, n1], axis=1)
    ge = jax.nn.gelu(agg)
    out = jnp.dot(ge, aW_ref[...], preferred_element_type=jnp.float32) + ab_ref[...]
    beta = beta_ref[0, 0]
    return beta * out + (1.0 - beta) * h_ref[...]


def _k2_body(h_ref, U_ref, den_ref, aW_ref, ab_ref, beta_ref, kqvW_ref, kqvb_ref,
             arelB_ref, mrelB_ref, qsc_ref, h1_ref, qT_ref, kvT_ref):
    h1 = _post_layer(h_ref, U_ref, den_ref, aW_ref, ab_ref, beta_ref)
    h1_ref[...] = h1
    _proj_stores(h1, kqvW_ref, kqvb_ref, arelB_ref, mrelB_ref, qsc_ref,
                 qT_ref, kvT_ref)


def _k3_body(h_ref, U_ref, den_ref, batch_ref,
             aW_ref, ab_ref, beta_ref,
             debW_ref, debb_ref, mWv_ref, mbv_ref, mWo_ref, mbo_ref,
             c1W_ref, c1b_ref, c2W_ref, c2b_ref,
             out_ref, gsum, cntm):
    i = pl.program_id(0)
    h2 = _post_layer(h_ref, U_ref, den_ref, aW_ref, ab_ref, beta_ref)
    b = batch_ref[...]
    oh = (b == lax.broadcasted_iota(jnp.int32, (BN, G), 1)).astype(jnp.float32)

    @pl.when(i == 0)
    def _():
        gsum[...] = jnp.zeros_like(gsum)
        cntm[...] = jnp.zeros_like(cntm)

    dn = (((0,), (0,)), ((), ()))
    gsum[...] += lax.dot_general(oh, h2, dn, preferred_element_type=jnp.float32,
                                 precision=lax.Precision.HIGHEST)
    cntm[...] += lax.dot_general(oh, jnp.ones_like(h2), dn,
                                 preferred_element_type=jnp.float32,
                                 precision=lax.Precision.HIGHEST)

    @pl.when(i == GRID - 1)
    def _():
        ge = gsum[...] / jnp.maximum(cntm[...], 1.0)
        g = jnp.dot(ge, debW_ref[...], preferred_element_type=jnp.float32) + debb_ref[...]
        attn = jnp.dot(g, mWv_ref[...], preferred_element_type=jnp.float32) + mbv_ref[...]
        attn = jnp.dot(attn, mWo_ref[...], preferred_element_type=jnp.float32) + mbo_ref[...]
        final = jnp.concatenate([g, attn], axis=1)
        hdn = jnp.dot(final, c1W_ref[...], preferred_element_type=jnp.float32) + c1b_ref[...]
        hdn = jnp.maximum(hdn, 0.0)
        out_ref[...] = jnp.dot(hdn, c2W_ref[...], preferred_element_type=jnp.float32) + c2b_ref[...]


def _edge_body(comb_hbm, dstr_hbm, qT_hbm, kvT_hbm, z32_hbm, z1_hbm,
               U_out, den_out,
               idx_v0, idx_v1, dstr_v0, dstr_v1,
               q_rows0, q_rows1, kv_rows0, kv_rows1,
               msg_v0, msg_v1, s_buf0, s_buf1,
               U_sh, den_sh,
               sem_i0, sem_i1, sem_d0, sem_d1, sem_q0, sem_q1,
               sem_kv0, sem_kv1, sem_u0, sem_u1, sem_n0, sem_n1):
    c = lax.axis_index("c")
    s = lax.axis_index("s")

    pltpu.sync_copy(z32_hbm, U_sh.at[pl.ds(s * NPT, NPT)])
    pltpu.sync_copy(z1_hbm, den_sh.at[pl.ds(s * NPT, NPT)])
    plsc.subcore_barrier()

    idx = (idx_v0, idx_v1)
    dstr = (dstr_v0, dstr_v1)
    qr = (q_rows0, q_rows1)
    kvr = (kv_rows0, kv_rows1)
    msg = (msg_v0, msg_v1)
    sb = (s_buf0, s_buf1)
    sem_i = (sem_i0, sem_i1)
    sem_d = (sem_d0, sem_d1)
    sem_q = (sem_q0, sem_q1)
    sem_kv = (sem_kv0, sem_kv1)
    sem_u = (sem_u0, sem_u1)
    sem_n = (sem_n0, sem_n1)

    cbase = c * ((E // CHUNK) * 2 * CHUNK)
    ebase = s * EPT

    def issue_idx(ch, p):
        off = cbase + (s * NCHUNK + ch) * (2 * CHUNK)
        pltpu.async_copy(comb_hbm.at[pl.ds(off, 2 * CHUNK)], idx[p], sem_i[p])
        pltpu.async_copy(dstr_hbm.at[pl.ds(ebase + ch * CHUNK, CHUNK)],
                         dstr[p], sem_d[p])

    def wait_idx(p):
        pltpu.make_async_copy(comb_hbm.at[pl.ds(0, 2 * CHUNK)], idx[p], sem_i[p]).wait()
        pltpu.make_async_copy(dstr_hbm.at[pl.ds(0, CHUNK)], dstr[p], sem_d[p]).wait()

    def issue_gathers(p):
        pltpu.async_copy(kvT_hbm.at[idx[p].at[pl.ds(0, CHUNK)]], kvr[p], sem_kv[p])
        pltpu.async_copy(qT_hbm.at[idx[p].at[pl.ds(CHUNK, CHUNK)]], qr[p], sem_q[p])

    def wait_gathers(p):
        pltpu.make_async_copy(qT_hbm.at[pl.ds(0, CHUNK)], qr[p], sem_q[p]).wait()
        pltpu.make_async_copy(kvT_hbm.at[pl.ds(0, CHUNK)], kvr[p], sem_kv[p]).wait()

    def issue_scatter(p):
        pltpu.async_copy(msg[p], U_sh.at[dstr[p]], sem_u[p], add=True)
        pltpu.async_copy(sb[p], den_sh.at[dstr[p]], sem_n[p], add=True)

    def wait_scatter(p):
        pltpu.make_async_copy(msg[p], U_sh.at[pl.ds(0, CHUNK)], sem_u[p]).wait()
        pltpu.make_async_copy(sb[p], den_sh.at[pl.ds(0, CHUNK)], sem_n[p]).wait()

    def compute(p):
        lane_ids = lax.iota(jnp.int32, LANES)

        def grp_body(gi, carry2):
            base = gi * LANES
            s_vec = jnp.zeros((LANES,), jnp.float32)
            for j in range(LANES):
                e = base + j
                q0 = qr[p][e, pl.ds(0, LANES)]
                q1 = qr[p][e, pl.ds(LANES, LANES)]
                k0 = kvr[p][e, pl.ds(0, LANES)]
                k1 = kvr[p][e, pl.ds(LANES, LANES)]
                pr = q0 * k0 + q1 * k1
                d_s = jnp.sum(pr)
                esv = jnp.exp(jnp.broadcast_to(d_s, (LANES,)))
                s_vec = jnp.where(lane_ids == j, esv, s_vec)
                v0 = kvr[p][e, pl.ds(2 * LANES, LANES)]
                v1 = kvr[p][e, pl.ds(3 * LANES, LANES)]
                msg[p][e, pl.ds(0, LANES)] = esv * v0
                msg[p][e, pl.ds(LANES, LANES)] = esv * v1
            sb[p][pl.ds(base, LANES)] = s_vec
            return carry2
        lax.fori_loop(0, CHUNK // LANES, grp_body, 0)

    issue_idx(0, 0)

    def chunk_iter(i, carry):
        for p in range(2):
            np_ = 1 - p

            @pl.when(lax.rem(i, 2) == p)
            def _():
                wait_idx(p)
                issue_gathers(p)

                @pl.when(i >= 1)
                def _():
                    wait_scatter(np_)

                @pl.when(i + 1 < NCHUNK)
                def _():
                    issue_idx(i + 1, np_)

                wait_gathers(p)
                compute(p)
                issue_scatter(p)
        return carry

    lax.fori_loop(0, NCHUNK, chunk_iter, 0)
    wait_scatter((NCHUNK - 1) % 2)
    plsc.subcore_barrier()

    obase = c * N_PAD + s * NPT
    pltpu.sync_copy(U_sh.at[pl.ds(s * NPT, NPT)],
                    U_out.at[pl.ds(obase, NPT)])
    pltpu.sync_copy(den_sh.at[pl.ds(s * NPT, NPT)],
                    den_out.at[pl.ds(obase, NPT)])


def _make_edge_phase():
    mesh = plsc.VectorSubcoreMesh(core_axis_name="c", subcore_axis_name="s",
                                  num_cores=NC, num_subcores=NS)
    dma = pltpu.SemaphoreType.DMA
    return pl.kernel(
        _edge_body,
        out_type=(jax.ShapeDtypeStruct((H * N_PAD, DH), jnp.float32),
                  jax.ShapeDtypeStruct((H * N_PAD,), jnp.float32)),
        mesh=mesh,
        compiler_params=pltpu.CompilerParams(needs_layout_passes=False,
                                             use_tc_tiling_on_sc=False),
        scratch_types=[
            pltpu.VMEM((2 * CHUNK,), jnp.int32),
            pltpu.VMEM((2 * CHUNK,), jnp.int32),
            pltpu.VMEM((CHUNK,), jnp.int32),
            pltpu.VMEM((CHUNK,), jnp.int32),
            pltpu.VMEM((CHUNK, DH), jnp.float32),
            pltpu.VMEM((CHUNK, DH), jnp.float32),
            pltpu.VMEM((CHUNK, 2 * DH), jnp.float32),
            pltpu.VMEM((CHUNK, 2 * DH), jnp.float32),
            pltpu.VMEM((CHUNK, DH), jnp.float32),
            pltpu.VMEM((CHUNK, DH), jnp.float32),
            pltpu.VMEM((CHUNK,), jnp.float32),
            pltpu.VMEM((CHUNK,), jnp.float32),
            pltpu.VMEM_SHARED((N_PAD, DH), jnp.float32),
            pltpu.VMEM_SHARED((N_PAD,), jnp.float32),
            dma, dma, dma, dma, dma, dma, dma, dma, dma, dma, dma, dma,
        ],
    )


def _block_diag2(a):
    z = jnp.zeros((DH, DH), jnp.float32)
    return jnp.block([[a[0], z], [z, a[1]]])


def _fold_layer(p, l):
    kqvW = jnp.concatenate([p['kW%d' % l], p['qW%d' % l], p['vW%d' % l]], axis=1)
    kqvb = jnp.concatenate([p['kb%d' % l], p['qb%d' % l], p['vb%d' % l]])[None, :]
    arelB = _block_diag2(p['a_rel%d' % l])
    mrelB = _block_diag2(p['m_rel%d' % l])
    scale = p['p_rel%d' % l] / jnp.sqrt(float(DH))
    qsc = jnp.repeat(scale, DH)[None, :]
    beta = jax.nn.sigmoid(p['skip%d' % l]).reshape(1, 1)
    return kqvW, kqvb, arelB, mrelB, qsc, beta


def kernel(x, edge_index, batch, params):
    p = params
    src = edge_index[0]
    dst = edge_index[1]

    kqvW0, kqvb0, arelB0, mrelB0, qsc0, beta0 = _fold_layer(p, 0)
    kqvW1, kqvb1, arelB1, mrelB1, qsc1, beta1 = _fold_layer(p, 1)

    full = lambda shape: pl.BlockSpec(shape, lambda i: (0,) * len(shape))

    h0, qT0, kvT0 = pl.pallas_call(
        _k1_body,
        grid=(GRID,),
        in_specs=[
            pl.BlockSpec((BN, IN_DIM), lambda i: (i, 0)),
            full((IN_DIM, HID)),
            full((1, HID)),
            full((HID, 3 * HID)),
            full((1, 3 * HID)),
            full((HID, HID)),
            full((HID, HID)),
            full((1, HID)),
        ],
        out_specs=[
            pl.BlockSpec((BN, HID), lambda i: (i, 0)),
            pl.BlockSpec((H, BN, DH), lambda i: (0, i, 0)),
            pl.BlockSpec((H, BN, 2 * DH), lambda i: (0, i, 0)),
        ],
        out_shape=[
            jax.ShapeDtypeStruct((N, HID), jnp.float32),
            jax.ShapeDtypeStruct((H, N, DH), jnp.float32),
            jax.ShapeDtypeStruct((H, N, 2 * DH), jnp.float32),
        ],
    )(x, p['lin_W'], p['lin_b'][None, :], kqvW0, kqvb0, arelB0, mrelB0, qsc0)

    edge_phase = _make_edge_phase()
    z32 = jnp.zeros((NPT, DH), jnp.float32)
    z1 = jnp.zeros((NPT,), jnp.float32)

    sc = src.reshape(E // CHUNK, CHUNK)
    dc = dst.reshape(E // CHUNK, CHUNK)
    comb = jnp.stack([jnp.concatenate([sc, dc], axis=1),
                      jnp.concatenate([sc + N, dc + N], axis=1)]).reshape(-1)

    U0, den0 = edge_phase(comb, dst, qT0.reshape(H * N, DH),
                          kvT0.reshape(H * N, 2 * DH), z32, z1)

    h1, qT1, kvT1 = pl.pallas_call(
        _k2_body,
        grid=(GRID,),
        in_specs=[
            pl.BlockSpec((BN, HID), lambda i: (i, 0)),
            pl.BlockSpec((H, BN, DH), lambda i: (0, i, 0)),
            pl.BlockSpec((H, BN, 1), lambda i: (0, i, 0)),
            full((HID, HID)),
            full((1, HID)),
            full((1, 1)),
            full((HID, 3 * HID)),
            full((1, 3 * HID)),
            full((HID, HID)),
            full((HID, HID)),
            full((1, HID)),
        ],
        out_specs=[
            pl.BlockSpec((BN, HID), lambda i: (i, 0)),
            pl.BlockSpec((H, BN, DH), lambda i: (0, i, 0)),
            pl.BlockSpec((H, BN, 2 * DH), lambda i: (0, i, 0)),
        ],
        out_shape=[
            jax.ShapeDtypeStruct((N, HID), jnp.float32),
            jax.ShapeDtypeStruct((H, N, DH), jnp.float32),
            jax.ShapeDtypeStruct((H, N, 2 * DH), jnp.float32),
        ],
    )(h0, U0.reshape(H, N_PAD, DH), den0.reshape(H, N_PAD, 1),
      p['aW0'], p['ab0'][None, :], beta0, kqvW1, kqvb1, arelB1, mrelB1, qsc1)

    U1, den1 = edge_phase(comb, dst, qT1.reshape(H * N, DH),
                          kvT1.reshape(H * N, 2 * DH), z32, z1)

    out = pl.pallas_call(
        _k3_body,
        grid=(GRID,),
        in_specs=[
            pl.BlockSpec((BN, HID), lambda i: (i, 0)),
            pl.BlockSpec((H, BN, DH), lambda i: (0, i, 0)),
            pl.BlockSpec((H, BN, 1), lambda i: (0, i, 0)),
            pl.BlockSpec((BN, 1), lambda i: (i, 0)),
            full((HID, HID)),
            full((1, HID)),
            full((1, 1)),
            full((HID, HID)),
            full((1, HID)),
            full((HID, HID)),
            full((1, HID)),
            full((HID, HID)),
            full((1, HID)),
            full((2 * HID, HID)),
            full((1, HID)),
            full((HID, 1)),
            full((1, 1)),
        ],
        out_specs=pl.BlockSpec((G, 1), lambda i: (0, 0)),
        out_shape=jax.ShapeDtypeStruct((G, 1), jnp.float32),
        scratch_shapes=[
            pltpu.VMEM((G, HID), jnp.float32),
            pltpu.VMEM((G, HID), jnp.float32),
        ],
    )(h1, U1.reshape(H, N_PAD, DH), den1.reshape(H, N_PAD, 1), batch.reshape(N, 1),
      p['aW1'], p['ab1'][None, :], beta1,
      p['deb_W'], p['deb_b'][None, :],
      p['mha_Wv'], p['mha_bv'][None, :],
      p['mha_Wo'], p['mha_bo'][None, :],
      p['c1_W'], p['c1_b'][None, :],
      p['c2_W'], p['c2_b'][None, :])

    return out


# speedup vs baseline: 1.6854x; 1.6854x over previous
import functools

import jax
import jax.numpy as jnp
from jax import lax
from jax.experimental import pallas as pl
from jax.experimental.pallas import tpu as pltpu
from jax.experimental.pallas import tpu_sc as plsc

N = 50000
E = 800000
IN_DIM = 771
HID = 64
H = 2
DH = 32
G = 64

NC = 2
NS = 16
LANES = 16

EPT = E // NS
CHUNK = 80
NCHUNK = EPT // CHUNK
NPT = 3128
N_PAD = NPT * NS

BN = 2000
GRID = N // BN


def _proj_stores(h1, kqvW_ref, kqvb_ref, arelB_ref, mrelB_ref, qsc_ref,
                 qT_ref, kvT_ref):
    kqv = jnp.dot(h1, kqvW_ref[...], preferred_element_type=jnp.float32) + kqvb_ref[...]
    k = jnp.dot(kqv[:, 0:HID], arelB_ref[...], preferred_element_type=jnp.float32)
    q = kqv[:, HID:2 * HID] * qsc_ref[...]
    v = jnp.dot(kqv[:, 2 * HID:3 * HID], mrelB_ref[...], preferred_element_type=jnp.float32)
    qT_ref[0] = q[:, 0:DH]
    qT_ref[1] = q[:, DH:2 * DH]
    kvT_ref[0, :, 0:DH] = k[:, 0:DH]
    kvT_ref[0, :, DH:2 * DH] = v[:, 0:DH]
    kvT_ref[1, :, 0:DH] = k[:, DH:2 * DH]
    kvT_ref[1, :, DH:2 * DH] = v[:, DH:2 * DH]


def _k1_body(x_ref, w_ref, b_ref, kqvW_ref, kqvb_ref, arelB_ref, mrelB_ref,
             qsc_ref, h_ref, qT_ref, kvT_ref):
    h = jnp.dot(x_ref[...], w_ref[...], preferred_element_type=jnp.float32)
    h = jnp.maximum(h + b_ref[...], 0.0)
    h_ref[...] = h
    _proj_stores(h, kqvW_ref, kqvb_ref, arelB_ref, mrelB_ref, qsc_ref,
                 qT_ref, kvT_ref)


def _post_layer(h_ref, U_ref, den_ref, aW_ref, ab_ref, beta_ref):
    n0 = U_ref[0] / (den_ref[0] + 1e-16)
    n1 = U_ref[1] / (den_ref[1] + 1e-16)
    agg = jnp.concatenate([n0, n1], axis=1)
    ge = jax.nn.gelu(agg)
    out = jnp.dot(ge, aW_ref[...], preferred_element_type=jnp.float32) + ab_ref[...]
    beta = beta_ref[0, 0]
    return beta * out + (1.0 - beta) * h_ref[...]


def _k2_body(h_ref, U_ref, den_ref, aW_ref, ab_ref, beta_ref, kqvW_ref, kqvb_ref,
             arelB_ref, mrelB_ref, qsc_ref, h1_ref, qT_ref, kvT_ref):
    h1 = _post_layer(h_ref, U_ref, den_ref, aW_ref, ab_ref, beta_ref)
    h1_ref[...] = h1
    _proj_stores(h1, kqvW_ref, kqvb_ref, arelB_ref, mrelB_ref, qsc_ref,
                 qT_ref, kvT_ref)


def _k3_body(h_ref, U_ref, den_ref, batch_ref,
             aW_ref, ab_ref, beta_ref,
             debW_ref, debb_ref, mWv_ref, mbv_ref, mWo_ref, mbo_ref,
             c1W_ref, c1b_ref, c2W_ref, c2b_ref,
             out_ref, gsum, cntm):
    i = pl.program_id(0)
    h2 = _post_layer(h_ref, U_ref, den_ref, aW_ref, ab_ref, beta_ref)
    b = batch_ref[...]
    oh = (b == lax.broadcasted_iota(jnp.int32, (BN, G), 1)).astype(jnp.float32)

    @pl.when(i == 0)
    def _():
        gsum[...] = jnp.zeros_like(gsum)
        cntm[...] = jnp.zeros_like(cntm)

    dn = (((0,), (0,)), ((), ()))
    gsum[...] += lax.dot_general(oh, h2, dn, preferred_element_type=jnp.float32,
                                 precision=lax.Precision.HIGHEST)
    cntm[...] += lax.dot_general(oh, jnp.ones_like(h2), dn,
                                 preferred_element_type=jnp.float32,
                                 precision=lax.Precision.HIGHEST)

    @pl.when(i == GRID - 1)
    def _():
        ge = gsum[...] / jnp.maximum(cntm[...], 1.0)
        g = jnp.dot(ge, debW_ref[...], preferred_element_type=jnp.float32) + debb_ref[...]
        attn = jnp.dot(g, mWv_ref[...], preferred_element_type=jnp.float32) + mbv_ref[...]
        attn = jnp.dot(attn, mWo_ref[...], preferred_element_type=jnp.float32) + mbo_ref[...]
        final = jnp.concatenate([g, attn], axis=1)
        hdn = jnp.dot(final, c1W_ref[...], preferred_element_type=jnp.float32) + c1b_ref[...]
        hdn = jnp.maximum(hdn, 0.0)
        out_ref[...] = jnp.dot(hdn, c2W_ref[...], preferred_element_type=jnp.float32) + c2b_ref[...]


def _edge_body(comb_hbm, dstr_hbm, qT_hbm, kvT_hbm, z32_hbm, z1_hbm,
               U_out, den_out,
               idx_v0, idx_v1, dstr_v0, dstr_v1,
               q_rows0, q_rows1, kv_rows0, kv_rows1,
               msg_v0, msg_v1, s_buf0, s_buf1,
               U_sh, den_sh,
               sem_i0, sem_i1, sem_d0, sem_d1, sem_q0, sem_q1,
               sem_kv0, sem_kv1, sem_u0, sem_u1, sem_n0, sem_n1):
    c = lax.axis_index("c")
    s = lax.axis_index("s")

    pltpu.sync_copy(z32_hbm, U_sh.at[pl.ds(s * NPT, NPT)])
    pltpu.sync_copy(z1_hbm, den_sh.at[pl.ds(s * NPT, NPT)])
    plsc.subcore_barrier()

    idx = (idx_v0, idx_v1)
    dstr = (dstr_v0, dstr_v1)
    qr = (q_rows0, q_rows1)
    kvr = (kv_rows0, kv_rows1)
    msg = (msg_v0, msg_v1)
    sb = (s_buf0, s_buf1)
    sem_i = (sem_i0, sem_i1)
    sem_d = (sem_d0, sem_d1)
    sem_q = (sem_q0, sem_q1)
    sem_kv = (sem_kv0, sem_kv1)
    sem_u = (sem_u0, sem_u1)
    sem_n = (sem_n0, sem_n1)

    cbase = c * ((E // CHUNK) * 2 * CHUNK)
    ebase = s * EPT

    def issue_idx(ch, p):
        off = cbase + (s * NCHUNK + ch) * (2 * CHUNK)
        pltpu.async_copy(comb_hbm.at[pl.ds(off, 2 * CHUNK)], idx[p], sem_i[p])
        pltpu.async_copy(dstr_hbm.at[pl.ds(ebase + ch * CHUNK, CHUNK)],
                         dstr[p], sem_d[p])

    def wait_idx(p):
        pltpu.make_async_copy(comb_hbm.at[pl.ds(0, 2 * CHUNK)], idx[p], sem_i[p]).wait()
        pltpu.make_async_copy(dstr_hbm.at[pl.ds(0, CHUNK)], dstr[p], sem_d[p]).wait()

    def issue_gathers(p):
        pltpu.async_copy(kvT_hbm.at[idx[p].at[pl.ds(0, CHUNK)]], kvr[p], sem_kv[p])
        pltpu.async_copy(qT_hbm.at[idx[p].at[pl.ds(CHUNK, CHUNK)]], qr[p], sem_q[p])

    def wait_gathers(p):
        pltpu.make_async_copy(qT_hbm.at[pl.ds(0, CHUNK)], qr[p], sem_q[p]).wait()
        pltpu.make_async_copy(kvT_hbm.at[pl.ds(0, CHUNK)], kvr[p], sem_kv[p]).wait()

    def issue_scatter(p):
        pltpu.async_copy(msg[p], U_sh.at[dstr[p]], sem_u[p], add=True)
        pltpu.async_copy(sb[p], den_sh.at[dstr[p]], sem_n[p], add=True)

    def wait_scatter(p):
        pltpu.make_async_copy(msg[p], U_sh.at[pl.ds(0, CHUNK)], sem_u[p]).wait()
        pltpu.make_async_copy(sb[p], den_sh.at[pl.ds(0, CHUNK)], sem_n[p]).wait()

    def compute(p):
        lane_ids = lax.iota(jnp.int32, LANES)

        for gi in range(CHUNK // LANES):
            base = gi * LANES
            s_vec = jnp.zeros((LANES,), jnp.float32)
            for j in range(LANES):
                e = base + j
                q0 = qr[p][e, pl.ds(0, LANES)]
                q1 = qr[p][e, pl.ds(LANES, LANES)]
                k0 = kvr[p][e, pl.ds(0, LANES)]
                k1 = kvr[p][e, pl.ds(LANES, LANES)]
                pr = q0 * k0 + q1 * k1
                d_s = jnp.sum(pr)
                s_vec = jnp.where(lane_ids == j, jnp.broadcast_to(d_s, (LANES,)), s_vec)
            es_vec = jnp.exp(s_vec)
            sb[p][pl.ds(base, LANES)] = es_vec
            for j in range(LANES):
                e = base + j
                esv = jnp.broadcast_to(es_vec[j], (LANES,))
                v0 = kvr[p][e, pl.ds(2 * LANES, LANES)]
                v1 = kvr[p][e, pl.ds(3 * LANES, LANES)]
                msg[p][e, pl.ds(0, LANES)] = esv * v0
                msg[p][e, pl.ds(LANES, LANES)] = esv * v1

    issue_idx(0, 0)

    def chunk_iter(i, carry):
        for p in range(2):
            np_ = 1 - p

            @pl.when(lax.rem(i, 2) == p)
            def _():
                wait_idx(p)
                issue_gathers(p)

                @pl.when(i >= 1)
                def _():
                    wait_scatter(np_)

                @pl.when(i + 1 < NCHUNK)
                def _():
                    issue_idx(i + 1, np_)

                wait_gathers(p)
                compute(p)
                issue_scatter(p)
        return carry

    lax.fori_loop(0, NCHUNK, chunk_iter, 0)
    wait_scatter((NCHUNK - 1) % 2)
    plsc.subcore_barrier()

    obase = c * N_PAD + s * NPT
    pltpu.sync_copy(U_sh.at[pl.ds(s * NPT, NPT)],
                    U_out.at[pl.ds(obase, NPT)])
    pltpu.sync_copy(den_sh.at[pl.ds(s * NPT, NPT)],
                    den_out.at[pl.ds(obase, NPT)])


def _make_edge_phase():
    mesh = plsc.VectorSubcoreMesh(core_axis_name="c", subcore_axis_name="s",
                                  num_cores=NC, num_subcores=NS)
    dma = pltpu.SemaphoreType.DMA
    return pl.kernel(
        _edge_body,
        out_type=(jax.ShapeDtypeStruct((H * N_PAD, DH), jnp.float32),
                  jax.ShapeDtypeStruct((H * N_PAD,), jnp.float32)),
        mesh=mesh,
        compiler_params=pltpu.CompilerParams(needs_layout_passes=False,
                                             use_tc_tiling_on_sc=False),
        scratch_types=[
            pltpu.VMEM((2 * CHUNK,), jnp.int32),
            pltpu.VMEM((2 * CHUNK,), jnp.int32),
            pltpu.VMEM((CHUNK,), jnp.int32),
            pltpu.VMEM((CHUNK,), jnp.int32),
            pltpu.VMEM((CHUNK, DH), jnp.float32),
            pltpu.VMEM((CHUNK, DH), jnp.float32),
            pltpu.VMEM((CHUNK, 2 * DH), jnp.float32),
            pltpu.VMEM((CHUNK, 2 * DH), jnp.float32),
            pltpu.VMEM((CHUNK, DH), jnp.float32),
            pltpu.VMEM((CHUNK, DH), jnp.float32),
            pltpu.VMEM((CHUNK,), jnp.float32),
            pltpu.VMEM((CHUNK,), jnp.float32),
            pltpu.VMEM_SHARED((N_PAD, DH), jnp.float32),
            pltpu.VMEM_SHARED((N_PAD,), jnp.float32),
            dma, dma, dma, dma, dma, dma, dma, dma, dma, dma, dma, dma,
        ],
    )


def _block_diag2(a):
    z = jnp.zeros((DH, DH), jnp.float32)
    return jnp.block([[a[0], z], [z, a[1]]])


def _fold_layer(p, l):
    kqvW = jnp.concatenate([p['kW%d' % l], p['qW%d' % l], p['vW%d' % l]], axis=1)
    kqvb = jnp.concatenate([p['kb%d' % l], p['qb%d' % l], p['vb%d' % l]])[None, :]
    arelB = _block_diag2(p['a_rel%d' % l])
    mrelB = _block_diag2(p['m_rel%d' % l])
    scale = p['p_rel%d' % l] / jnp.sqrt(float(DH))
    qsc = jnp.repeat(scale, DH)[None, :]
    beta = jax.nn.sigmoid(p['skip%d' % l]).reshape(1, 1)
    return kqvW, kqvb, arelB, mrelB, qsc, beta


def kernel(x, edge_index, batch, params):
    p = params
    src = edge_index[0]
    dst = edge_index[1]

    kqvW0, kqvb0, arelB0, mrelB0, qsc0, beta0 = _fold_layer(p, 0)
    kqvW1, kqvb1, arelB1, mrelB1, qsc1, beta1 = _fold_layer(p, 1)

    full = lambda shape: pl.BlockSpec(shape, lambda i: (0,) * len(shape))

    h0, qT0, kvT0 = pl.pallas_call(
        _k1_body,
        grid=(GRID,),
        in_specs=[
            pl.BlockSpec((BN, IN_DIM), lambda i: (i, 0)),
            full((IN_DIM, HID)),
            full((1, HID)),
            full((HID, 3 * HID)),
            full((1, 3 * HID)),
            full((HID, HID)),
            full((HID, HID)),
            full((1, HID)),
        ],
        out_specs=[
            pl.BlockSpec((BN, HID), lambda i: (i, 0)),
            pl.BlockSpec((H, BN, DH), lambda i: (0, i, 0)),
            pl.BlockSpec((H, BN, 2 * DH), lambda i: (0, i, 0)),
        ],
        out_shape=[
            jax.ShapeDtypeStruct((N, HID), jnp.float32),
            jax.ShapeDtypeStruct((H, N, DH), jnp.float32),
            jax.ShapeDtypeStruct((H, N, 2 * DH), jnp.float32),
        ],
    )(x, p['lin_W'], p['lin_b'][None, :], kqvW0, kqvb0, arelB0, mrelB0, qsc0)

    edge_phase = _make_edge_phase()
    z32 = jnp.zeros((NPT, DH), jnp.float32)
    z1 = jnp.zeros((NPT,), jnp.float32)

    sc = src.reshape(E // CHUNK, CHUNK)
    dc = dst.reshape(E // CHUNK, CHUNK)
    comb = jnp.stack([jnp.concatenate([sc, dc], axis=1),
                      jnp.concatenate([sc + N, dc + N], axis=1)]).reshape(-1)

    U0, den0 = edge_phase(comb, dst, qT0.reshape(H * N, DH),
                          kvT0.reshape(H * N, 2 * DH), z32, z1)

    h1, qT1, kvT1 = pl.pallas_call(
        _k2_body,
        grid=(GRID,),
        in_specs=[
            pl.BlockSpec((BN, HID), lambda i: (i, 0)),
            pl.BlockSpec((H, BN, DH), lambda i: (0, i, 0)),
            pl.BlockSpec((H, BN, 1), lambda i: (0, i, 0)),
            full((HID, HID)),
            full((1, HID)),
            full((1, 1)),
            full((HID, 3 * HID)),
            full((1, 3 * HID)),
            full((HID, HID)),
            full((HID, HID)),
            full((1, HID)),
        ],
        out_specs=[
            pl.BlockSpec((BN, HID), lambda i: (i, 0)),
            pl.BlockSpec((H, BN, DH), lambda i: (0, i, 0)),
            pl.BlockSpec((H, BN, 2 * DH), lambda i: (0, i, 0)),
        ],
        out_shape=[
            jax.ShapeDtypeStruct((N, HID), jnp.float32),
            jax.ShapeDtypeStruct((H, N, DH), jnp.float32),
            jax.ShapeDtypeStruct((H, N, 2 * DH), jnp.float32),
        ],
    )(h0, U0.reshape(H, N_PAD, DH), den0.reshape(H, N_PAD, 1),
      p['aW0'], p['ab0'][None, :], beta0, kqvW1, kqvb1, arelB1, mrelB1, qsc1)

    U1, den1 = edge_phase(comb, dst, qT1.reshape(H * N, DH),
                          kvT1.reshape(H * N, 2 * DH), z32, z1)

    out = pl.pallas_call(
        _k3_body,
        grid=(GRID,),
        in_specs=[
            pl.BlockSpec((BN, HID), lambda i: (i, 0)),
            pl.BlockSpec((H, BN, DH), lambda i: (0, i, 0)),
            pl.BlockSpec((H, BN, 1), lambda i: (0, i, 0)),
            pl.BlockSpec((BN, 1), lambda i: (i, 0)),
            full((HID, HID)),
            full((1, HID)),
            full((1, 1)),
            full((HID, HID)),
            full((1, HID)),
            full((HID, HID)),
            full((1, HID)),
            full((HID, HID)),
            full((1, HID)),
            full((2 * HID, HID)),
            full((1, HID)),
            full((HID, 1)),
            full((1, 1)),
        ],
        out_specs=pl.BlockSpec((G, 1), lambda i: (0, 0)),
        out_shape=jax.ShapeDtypeStruct((G, 1), jnp.float32),
        scratch_shapes=[
            pltpu.VMEM((G, HID), jnp.float32),
            pltpu.VMEM((G, HID), jnp.float32),
        ],
    )(h1, U1.reshape(H, N_PAD, DH), den1.reshape(H, N_PAD, 1), batch.reshape(N, 1),
      p['aW1'], p['ab1'][None, :], beta1,
      p['deb_W'], p['deb_b'][None, :],
      p['mha_Wv'], p['mha_bv'][None, :],
      p['mha_Wo'], p['mha_bo'][None, :],
      p['c1_W'], p['c1_b'][None, :],
      p['c2_W'], p['c2_b'][None, :])

    return out


# speedup vs baseline: 2.0771x; 1.2324x over previous
import functools

import jax
import jax.numpy as jnp
from jax import lax
from jax.experimental import pallas as pl
from jax.experimental.pallas import tpu as pltpu
from jax.experimental.pallas import tpu_sc as plsc

N = 50000
E = 800000
IN_DIM = 771
HID = 64
H = 2
DH = 32
G = 64

NC = 2
NS = 16
LANES = 16

EPT = E // NS
CHUNK = 80
NCHUNK = EPT // CHUNK
NPT = 3128
N_PAD = NPT * NS

BN = 2000
GRID = N // BN


def _proj_stores(h1, kqvW_ref, kqvb_ref, arelB_ref, mrelB_ref, qsc_ref,
                 qT_ref, kvT_ref):
    kqv = jnp.dot(h1, kqvW_ref[...], preferred_element_type=jnp.float32) + kqvb_ref[...]
    k = jnp.dot(kqv[:, 0:HID], arelB_ref[...], preferred_element_type=jnp.float32)
    q = kqv[:, HID:2 * HID] * qsc_ref[...]
    v = jnp.dot(kqv[:, 2 * HID:3 * HID], mrelB_ref[...], preferred_element_type=jnp.float32)
    qT_ref[0] = q[:, 0:DH]
    qT_ref[1] = q[:, DH:2 * DH]
    kvT_ref[0, :, 0:DH] = k[:, 0:DH]
    kvT_ref[0, :, DH:2 * DH] = v[:, 0:DH]
    kvT_ref[1, :, 0:DH] = k[:, DH:2 * DH]
    kvT_ref[1, :, DH:2 * DH] = v[:, DH:2 * DH]


def _k1_body(x_ref, w_ref, b_ref, kqvW_ref, kqvb_ref, arelB_ref, mrelB_ref,
             qsc_ref, h_ref, qT_ref, kvT_ref):
    h = jnp.dot(x_ref[...], w_ref[...], preferred_element_type=jnp.float32)
    h = jnp.maximum(h + b_ref[...], 0.0)
    h_ref[...] = h
    _proj_stores(h, kqvW_ref, kqvb_ref, arelB_ref, mrelB_ref, qsc_ref,
                 qT_ref, kvT_ref)


def _post_layer(h_ref, U_ref, den_ref, aW_ref, ab_ref, beta_ref):
    n0 = U_ref[0] / (den_ref[0] + 1e-16)
    n1 = U_ref[1] / (den_ref[1] + 1e-16)
    agg = jnp.concatenate([n0, n1], axis=1)
    ge = jax.nn.gelu(agg)
    out = jnp.dot(ge, aW_ref[...], preferred_element_type=jnp.float32) + ab_ref[...]
    beta = beta_ref[0, 0]
    return beta * out + (1.0 - beta) * h_ref[...]


def _k2_body(h_ref, U_ref, den_ref, aW_ref, ab_ref, beta_ref, kqvW_ref, kqvb_ref,
             arelB_ref, mrelB_ref, qsc_ref, h1_ref, qT_ref, kvT_ref):
    h1 = _post_layer(h_ref, U_ref, den_ref, aW_ref, ab_ref, beta_ref)
    h1_ref[...] = h1
    _proj_stores(h1, kqvW_ref, kqvb_ref, arelB_ref, mrelB_ref, qsc_ref,
                 qT_ref, kvT_ref)


def _k3_body(h_ref, U_ref, den_ref, batch_ref,
             aW_ref, ab_ref, beta_ref,
             debW_ref, debb_ref, mWv_ref, mbv_ref, mWo_ref, mbo_ref,
             c1W_ref, c1b_ref, c2W_ref, c2b_ref,
             out_ref, gsum, cntm):
    i = pl.program_id(0)
    h2 = _post_layer(h_ref, U_ref, den_ref, aW_ref, ab_ref, beta_ref)
    b = batch_ref[...]
    oh = (b == lax.broadcasted_iota(jnp.int32, (BN, G), 1)).astype(jnp.float32)

    @pl.when(i == 0)
    def _():
        gsum[...] = jnp.zeros_like(gsum)
        cntm[...] = jnp.zeros_like(cntm)

    dn = (((0,), (0,)), ((), ()))
    gsum[...] += lax.dot_general(oh, h2, dn, preferred_element_type=jnp.float32,
                                 precision=lax.Precision.HIGHEST)
    cntm[...] += lax.dot_general(oh, jnp.ones_like(h2), dn,
                                 preferred_element_type=jnp.float32,
                                 precision=lax.Precision.HIGHEST)

    @pl.when(i == GRID - 1)
    def _():
        ge = gsum[...] / jnp.maximum(cntm[...], 1.0)
        g = jnp.dot(ge, debW_ref[...], preferred_element_type=jnp.float32) + debb_ref[...]
        attn = jnp.dot(g, mWv_ref[...], preferred_element_type=jnp.float32) + mbv_ref[...]
        attn = jnp.dot(attn, mWo_ref[...], preferred_element_type=jnp.float32) + mbo_ref[...]
        final = jnp.concatenate([g, attn], axis=1)
        hdn = jnp.dot(final, c1W_ref[...], preferred_element_type=jnp.float32) + c1b_ref[...]
        hdn = jnp.maximum(hdn, 0.0)
        out_ref[...] = jnp.dot(hdn, c2W_ref[...], preferred_element_type=jnp.float32) + c2b_ref[...]


def _edge_body(comb_hbm, dstr_hbm, qT_hbm, kvT_hbm, z32_hbm, z1_hbm,
               U_out, den_out,
               idx_v0, idx_v1, dstr_v0, dstr_v1,
               q_rows0, q_rows1, kv_rows0, kv_rows1,
               msg_v0, msg_v1, s_buf0, s_buf1,
               U_sh, den_sh,
               sem_i0, sem_i1, sem_d0, sem_d1, sem_q0, sem_q1,
               sem_kv0, sem_kv1, sem_u0, sem_u1, sem_n0, sem_n1):
    c = lax.axis_index("c")
    s = lax.axis_index("s")

    pltpu.sync_copy(z32_hbm, U_sh.at[pl.ds(s * NPT, NPT)])
    pltpu.sync_copy(z1_hbm, den_sh.at[pl.ds(s * NPT, NPT)])
    plsc.subcore_barrier()

    idx = (idx_v0, idx_v1)
    dstr = (dstr_v0, dstr_v1)
    qr = (q_rows0, q_rows1)
    kvr = (kv_rows0, kv_rows1)
    msg = (msg_v0, msg_v1)
    sb = (s_buf0, s_buf1)
    sem_i = (sem_i0, sem_i1)
    sem_d = (sem_d0, sem_d1)
    sem_q = (sem_q0, sem_q1)
    sem_kv = (sem_kv0, sem_kv1)
    sem_u = (sem_u0, sem_u1)
    sem_n = (sem_n0, sem_n1)

    cbase = c * ((E // CHUNK) * 2 * CHUNK)
    ebase = s * EPT

    def issue_comb(ch, p):
        off = cbase + (s * NCHUNK + ch) * (2 * CHUNK)
        pltpu.async_copy(comb_hbm.at[pl.ds(off, 2 * CHUNK)], idx[p], sem_i[p])

    def wait_comb(p):
        pltpu.make_async_copy(comb_hbm.at[pl.ds(0, 2 * CHUNK)], idx[p], sem_i[p]).wait()

    def issue_dstr(ch, p):
        pltpu.async_copy(dstr_hbm.at[pl.ds(ebase + ch * CHUNK, CHUNK)],
                         dstr[p], sem_d[p])

    def wait_dstr(p):
        pltpu.make_async_copy(dstr_hbm.at[pl.ds(0, CHUNK)], dstr[p], sem_d[p]).wait()

    def issue_gathers(p):
        pltpu.async_copy(kvT_hbm.at[idx[p].at[pl.ds(0, CHUNK)]], kvr[p], sem_kv[p])
        pltpu.async_copy(qT_hbm.at[idx[p].at[pl.ds(CHUNK, CHUNK)]], qr[p], sem_q[p])

    def wait_gathers(p):
        pltpu.make_async_copy(qT_hbm.at[pl.ds(0, CHUNK)], qr[p], sem_q[p]).wait()
        pltpu.make_async_copy(kvT_hbm.at[pl.ds(0, CHUNK)], kvr[p], sem_kv[p]).wait()

    def issue_scatter(p):
        pltpu.async_copy(msg[p], U_sh.at[dstr[p]], sem_u[p], add=True)
        pltpu.async_copy(sb[p], den_sh.at[dstr[p]], sem_n[p], add=True)

    def wait_scatter(p):
        pltpu.make_async_copy(msg[p], U_sh.at[pl.ds(0, CHUNK)], sem_u[p]).wait()
        pltpu.make_async_copy(sb[p], den_sh.at[pl.ds(0, CHUNK)], sem_n[p]).wait()

    def compute(p):
        lane_ids = lax.iota(jnp.int32, LANES)
        for gi in range(CHUNK // LANES):
            base = gi * LANES
            s_vec = jnp.zeros((LANES,), jnp.float32)
            for j in range(LANES):
                e = base + j
                q0 = qr[p][e, pl.ds(0, LANES)]
                q1 = qr[p][e, pl.ds(LANES, LANES)]
                k0 = kvr[p][e, pl.ds(0, LANES)]
                k1 = kvr[p][e, pl.ds(LANES, LANES)]
                pr = q0 * k0 + q1 * k1
                d_s = jnp.sum(pr)
                s_vec = jnp.where(lane_ids == j, jnp.broadcast_to(d_s, (LANES,)), s_vec)
            es_vec = jnp.exp(s_vec)
            sb[p][pl.ds(base, LANES)] = es_vec
            for j in range(LANES):
                e = base + j
                esv = jnp.broadcast_to(es_vec[j], (LANES,))
                v0 = kvr[p][e, pl.ds(2 * LANES, LANES)]
                v1 = kvr[p][e, pl.ds(3 * LANES, LANES)]
                msg[p][e, pl.ds(0, LANES)] = esv * v0
                msg[p][e, pl.ds(LANES, LANES)] = esv * v1

    issue_comb(0, 0)
    issue_comb(1, 1)
    wait_comb(0)
    issue_gathers(0)

    def chunk_iter(i, carry):
        for p in range(2):
            np_ = 1 - p

            @pl.when(lax.rem(i, 2) == p)
            def _():
                wait_gathers(p)

                @pl.when(i + 1 < NCHUNK)
                def _():
                    wait_comb(np_)
                    issue_gathers(np_)

                @pl.when(i + 2 < NCHUNK)
                def _():
                    issue_comb(i + 2, p)

                @pl.when(i >= 2)
                def _():
                    wait_scatter(p)

                issue_dstr(i, p)
                compute(p)
                wait_dstr(p)
                issue_scatter(p)
        return carry

    lax.fori_loop(0, NCHUNK, chunk_iter, 0)
    wait_scatter((NCHUNK - 2) % 2)
    wait_scatter((NCHUNK - 1) % 2)
    plsc.subcore_barrier()

    obase = c * N_PAD + s * NPT
    pltpu.sync_copy(U_sh.at[pl.ds(s * NPT, NPT)],
                    U_out.at[pl.ds(obase, NPT)])
    pltpu.sync_copy(den_sh.at[pl.ds(s * NPT, NPT)],
                    den_out.at[pl.ds(obase, NPT)])


def _make_edge_phase():
    mesh = plsc.VectorSubcoreMesh(core_axis_name="c", subcore_axis_name="s",
                                  num_cores=NC, num_subcores=NS)
    dma = pltpu.SemaphoreType.DMA
    return pl.kernel(
        _edge_body,
        out_type=(jax.ShapeDtypeStruct((H * N_PAD, DH), jnp.float32),
                  jax.ShapeDtypeStruct((H * N_PAD,), jnp.float32)),
        mesh=mesh,
        compiler_params=pltpu.CompilerParams(needs_layout_passes=False,
                                             use_tc_tiling_on_sc=False),
        scratch_types=[
            pltpu.VMEM((2 * CHUNK,), jnp.int32),
            pltpu.VMEM((2 * CHUNK,), jnp.int32),
            pltpu.VMEM((CHUNK,), jnp.int32),
            pltpu.VMEM((CHUNK,), jnp.int32),
            pltpu.VMEM((CHUNK, DH), jnp.float32),
            pltpu.VMEM((CHUNK, DH), jnp.float32),
            pltpu.VMEM((CHUNK, 2 * DH), jnp.float32),
            pltpu.VMEM((CHUNK, 2 * DH), jnp.float32),
            pltpu.VMEM((CHUNK, DH), jnp.float32),
            pltpu.VMEM((CHUNK, DH), jnp.float32),
            pltpu.VMEM((CHUNK,), jnp.float32),
            pltpu.VMEM((CHUNK,), jnp.float32),
            pltpu.VMEM_SHARED((N_PAD, DH), jnp.float32),
            pltpu.VMEM_SHARED((N_PAD,), jnp.float32),
            dma, dma, dma, dma, dma, dma, dma, dma, dma, dma, dma, dma,
        ],
    )


def _block_diag2(a):
    z = jnp.zeros((DH, DH), jnp.float32)
    return jnp.block([[a[0], z], [z, a[1]]])


def _fold_layer(p, l):
    kqvW = jnp.concatenate([p['kW%d' % l], p['qW%d' % l], p['vW%d' % l]], axis=1)
    kqvb = jnp.concatenate([p['kb%d' % l], p['qb%d' % l], p['vb%d' % l]])[None, :]
    arelB = _block_diag2(p['a_rel%d' % l])
    mrelB = _block_diag2(p['m_rel%d' % l])
    scale = p['p_rel%d' % l] / jnp.sqrt(float(DH))
    qsc = jnp.repeat(scale, DH)[None, :]
    beta = jax.nn.sigmoid(p['skip%d' % l]).reshape(1, 1)
    return kqvW, kqvb, arelB, mrelB, qsc, beta


def kernel(x, edge_index, batch, params):
    p = params
    src = edge_index[0]
    dst = edge_index[1]

    kqvW0, kqvb0, arelB0, mrelB0, qsc0, beta0 = _fold_layer(p, 0)
    kqvW1, kqvb1, arelB1, mrelB1, qsc1, beta1 = _fold_layer(p, 1)

    full = lambda shape: pl.BlockSpec(shape, lambda i: (0,) * len(shape))

    h0, qT0, kvT0 = pl.pallas_call(
        _k1_body,
        grid=(GRID,),
        in_specs=[
            pl.BlockSpec((BN, IN_DIM), lambda i: (i, 0)),
            full((IN_DIM, HID)),
            full((1, HID)),
            full((HID, 3 * HID)),
            full((1, 3 * HID)),
            full((HID, HID)),
            full((HID, HID)),
            full((1, HID)),
        ],
        out_specs=[
            pl.BlockSpec((BN, HID), lambda i: (i, 0)),
            pl.BlockSpec((H, BN, DH), lambda i: (0, i, 0)),
            pl.BlockSpec((H, BN, 2 * DH), lambda i: (0, i, 0)),
        ],
        out_shape=[
            jax.ShapeDtypeStruct((N, HID), jnp.float32),
            jax.ShapeDtypeStruct((H, N, DH), jnp.float32),
            jax.ShapeDtypeStruct((H, N, 2 * DH), jnp.float32),
        ],
    )(x, p['lin_W'], p['lin_b'][None, :], kqvW0, kqvb0, arelB0, mrelB0, qsc0)

    edge_phase = _make_edge_phase()
    z32 = jnp.zeros((NPT, DH), jnp.float32)
    z1 = jnp.zeros((NPT,), jnp.float32)

    sc = src.reshape(E // CHUNK, CHUNK)
    dc = dst.reshape(E // CHUNK, CHUNK)
    comb = jnp.stack([jnp.concatenate([sc, dc], axis=1),
                      jnp.concatenate([sc + N, dc + N], axis=1)]).reshape(-1)

    U0, den0 = edge_phase(comb, dst, qT0.reshape(H * N, DH),
                          kvT0.reshape(H * N, 2 * DH), z32, z1)

    h1, qT1, kvT1 = pl.pallas_call(
        _k2_body,
        grid=(GRID,),
        in_specs=[
            pl.BlockSpec((BN, HID), lambda i: (i, 0)),
            pl.BlockSpec((H, BN, DH), lambda i: (0, i, 0)),
            pl.BlockSpec((H, BN, 1), lambda i: (0, i, 0)),
            full((HID, HID)),
            full((1, HID)),
            full((1, 1)),
            full((HID, 3 * HID)),
            full((1, 3 * HID)),
            full((HID, HID)),
            full((HID, HID)),
            full((1, HID)),
        ],
        out_specs=[
            pl.BlockSpec((BN, HID), lambda i: (i, 0)),
            pl.BlockSpec((H, BN, DH), lambda i: (0, i, 0)),
            pl.BlockSpec((H, BN, 2 * DH), lambda i: (0, i, 0)),
        ],
        out_shape=[
            jax.ShapeDtypeStruct((N, HID), jnp.float32),
            jax.ShapeDtypeStruct((H, N, DH), jnp.float32),
            jax.ShapeDtypeStruct((H, N, 2 * DH), jnp.float32),
        ],
    )(h0, U0.reshape(H, N_PAD, DH), den0.reshape(H, N_PAD, 1),
      p['aW0'], p['ab0'][None, :], beta0, kqvW1, kqvb1, arelB1, mrelB1, qsc1)

    U1, den1 = edge_phase(comb, dst, qT1.reshape(H * N, DH),
                          kvT1.reshape(H * N, 2 * DH), z32, z1)

    out = pl.pallas_call(
        _k3_body,
        grid=(GRID,),
        in_specs=[
            pl.BlockSpec((BN, HID), lambda i: (i, 0)),
            pl.BlockSpec((H, BN, DH), lambda i: (0, i, 0)),
            pl.BlockSpec((H, BN, 1), lambda i: (0, i, 0)),
            pl.BlockSpec((BN, 1), lambda i: (i, 0)),
            full((HID, HID)),
            full((1, HID)),
            full((1, 1)),
            full((HID, HID)),
            full((1, HID)),
            full((HID, HID)),
            full((1, HID)),
            full((HID, HID)),
            full((1, HID)),
            full((2 * HID, HID)),
            full((1, HID)),
            full((HID, 1)),
            full((1, 1)),
        ],
        out_specs=pl.BlockSpec((G, 1), lambda i: (0, 0)),
        out_shape=jax.ShapeDtypeStruct((G, 1), jnp.float32),
        scratch_shapes=[
            pltpu.VMEM((G, HID), jnp.float32),
            pltpu.VMEM((G, HID), jnp.float32),
        ],
    )(h1, U1.reshape(H, N_PAD, DH), den1.reshape(H, N_PAD, 1), batch.reshape(N, 1),
      p['aW1'], p['ab1'][None, :], beta1,
      p['deb_W'], p['deb_b'][None, :],
      p['mha_Wv'], p['mha_bv'][None, :],
      p['mha_Wo'], p['mha_bo'][None, :],
      p['c1_W'], p['c1_b'][None, :],
      p['c2_W'], p['c2_b'][None, :])

    return out
